# Initial kernel scaffold; baseline (speedup 1.0000x reference)
import jax
import jax.numpy as jnp
from jax.experimental import pallas as pl


def kernel(xyz0, xyz1, xyz2, init_feats, nei_inds0, nei_inds1, inv_neighbors0, inv_neighbors1, inv_k0, inv_k1, inv_idx0, inv_idx1, wn0_W, wn0_b, lin0_W, lin0_b, wn1_W, wn1_b, lin1_W, lin1_b):
    raise NotImplementedError("write your pallas kernel here")


# speedup vs baseline: 6.2845x; 6.2845x over previous
import functools

import jax
import jax.numpy as jnp
from jax import lax
from jax.experimental import pallas as pl
from jax.experimental.pallas import tpu as pltpu
from jax.experimental.pallas import tpu_sc as plsc

_NC = 2
_NS = 16
_NW = _NC * _NS
_CHUNK = 128
_K = 16
_W = 16


def _sc_gather_planar(planes, idx, n, bm, m):
    npl = len(planes)
    r = idx.shape[0]
    rpw = r // _NW
    nch = rpw // _CHUNK
    mesh = plsc.VectorSubcoreMesh(core_axis_name="c", subcore_axis_name="s")
    out_type = jax.ShapeDtypeStruct((r, 16), jnp.float32)
    scratch = (
        [pltpu.VMEM((n,), jnp.float32) for _ in planes]
        + [pltpu.VMEM((_CHUNK,), jnp.int32), pltpu.VMEM((_CHUNK, 16), jnp.float32)]
    )

    def body(*refs):
        plane_hbm = refs[:npl]
        idx_hbm = refs[npl]
        out_hbm = refs[npl + 1]
        plane_v = refs[npl + 2 : 2 * npl + 2]
        idx_v = refs[2 * npl + 2]
        fbuf = refs[2 * npl + 3]
        wid = lax.axis_index("s") * _NC + lax.axis_index("c")
        base = wid * rpw
        batch = lax.rem(base, bm) // m
        for p in range(npl):
            pltpu.sync_copy(plane_hbm[p].at[pl.ds(batch * n, n)], plane_v[p])
        iota16 = lax.iota(jnp.int32, 16)

        def step(c, carry):
            off = base + c * _CHUNK
            pltpu.sync_copy(idx_hbm.at[pl.ds(off, _CHUNK)], idx_v)
            for g in range(_CHUNK // 16):
                iv = idx_v[pl.ds(g * 16, 16)]
                rows = iota16 + (g * 16)
                for p in range(npl):
                    vals = plsc.load_gather(plane_v[p], [iv])
                    cols = jnp.full((16,), p, jnp.int32)
                    plsc.store_scatter(fbuf, [rows, cols], vals)
            pltpu.sync_copy(fbuf, out_hbm.at[pl.ds(off, _CHUNK)])
            return carry

        lax.fori_loop(0, nch, step, 0)

    fn = pl.kernel(
        body, out_type=out_type, mesh=mesh, scratch_types=scratch,
        compiler_params=pltpu.CompilerParams(needs_layout_passes=False),
    )
    return fn(*planes, idx)


def _sc_gather(tables, idx):
    nt = len(tables)
    r = idx.shape[0]
    rpw = r // _NW
    nch = rpw // _CHUNK
    mesh = plsc.VectorSubcoreMesh(core_axis_name="c", subcore_axis_name="s")
    out_type = tuple(
        jax.ShapeDtypeStruct((r, t.shape[1]), jnp.float32) for t in tables
    )
    scratch = (
        [pltpu.VMEM((_CHUNK,), jnp.int32)]
        + [pltpu.VMEM((_CHUNK, t.shape[1]), jnp.float32) for t in tables]
        + [pltpu.SemaphoreType.DMA]
    )

    def body(*refs):
        tabs = refs[:nt]
        idx_hbm = refs[nt]
        outs = refs[nt + 1 : 2 * nt + 1]
        idx_v = refs[2 * nt + 1]
        bufs = refs[2 * nt + 2 : 3 * nt + 2]
        sem = refs[-1]
        wid = lax.axis_index("s") * _NC + lax.axis_index("c")
        base = wid * rpw

        def step(j, carry):
            off = base + j * _CHUNK
            pltpu.sync_copy(idx_hbm.at[pl.ds(off, _CHUNK)], idx_v)
            cps = [
                pltpu.async_copy(tabs[t].at[idx_v], bufs[t], sem)
                for t in range(nt)
            ]
            for cp in cps:
                cp.wait()
            for t in range(nt):
                pltpu.sync_copy(bufs[t], outs[t].at[pl.ds(off, _CHUNK)])
            return carry

        lax.fori_loop(0, nch, step, 0)

    fn = pl.kernel(body, out_type=out_type, mesh=mesh, scratch_types=scratch)
    return list(fn(*tables, idx))


def _tc_layer(g_xyz, g_feat, sxyz, wn_W, wn_b, lin_W2, lin_b, cin, cout, mt,
              fused_feat_off=None):
    bm = sxyz.shape[0]
    grid = (bm // mt,)
    dx = g_xyz.shape[2]
    sep = g_feat is not None

    def body(*refs):
        if sep:
            gx_ref, gf_ref, sx_ref, wnw_ref, wnb_ref, w2_ref, b_ref, o_ref = refs
        else:
            gx_ref, sx_ref, wnw_ref, wnb_ref, w2_ref, b_ref, o_ref = refs
        sx = sx_ref[...]
        wnw = wnw_ref[...]
        wnb = wnb_ref[...]
        accs = [None] * _W
        for k in range(_K):
            gxk = gx_ref[k]
            rel = gxk[:, 0:3] - sx
            wk = (
                wnb
                + rel[:, 0:1] * wnw[0:1, :]
                + rel[:, 1:2] * wnw[1:2, :]
                + rel[:, 2:3] * wnw[2:3, :]
            )
            wk = jnp.maximum(wk, 0.0)
            if sep:
                fk = gf_ref[k]
            else:
                fk = gxk[:, fused_feat_off : fused_feat_off + cin]
            for w in range(_W):
                t = fk * wk[:, w : w + 1]
                accs[w] = t if accs[w] is None else accs[w] + t
        nf = jnp.concatenate(accs, axis=1)
        out = jnp.dot(nf, w2_ref[...], preferred_element_type=jnp.float32)
        o_ref[...] = jnp.maximum(out + b_ref[...], 0.0)

    in_specs = [pl.BlockSpec((_K, mt, dx), lambda i: (0, i, 0))]
    args = [g_xyz]
    if sep:
        in_specs.append(pl.BlockSpec((_K, mt, cin), lambda i: (0, i, 0)))
        args.append(g_feat)
    in_specs += [
        pl.BlockSpec((mt, 3), lambda i: (i, 0)),
        pl.BlockSpec((3, _W), lambda i: (0, 0)),
        pl.BlockSpec((1, _W), lambda i: (0, 0)),
        pl.BlockSpec((_W * cin, cout), lambda i: (0, 0)),
        pl.BlockSpec((1, cout), lambda i: (0, 0)),
    ]
    args += [sxyz, wn_W, wn_b.reshape(1, _W), lin_W2, lin_b.reshape(1, cout)]

    return pl.pallas_call(
        body,
        grid=grid,
        in_specs=in_specs,
        out_specs=pl.BlockSpec((mt, cout), lambda i: (i, 0)),
        out_shape=jax.ShapeDtypeStruct((bm, cout), jnp.float32),
        compiler_params=pltpu.CompilerParams(
            dimension_semantics=("arbitrary",)
        ),
    )(*args)


def _flat_idx(nei_inds, n):
    b = nei_inds.shape[0]
    off = (jnp.arange(b, dtype=jnp.int32) * n)[:, None, None]
    return (nei_inds + off).transpose(2, 0, 1).reshape(-1)


def _wmajor(lin_w, cin):
    cout = lin_w.shape[1]
    return lin_w.reshape(cin, _W, cout).transpose(1, 0, 2).reshape(_W * cin, cout)


def kernel(xyz0, xyz1, xyz2, init_feats, nei_inds0, nei_inds1,
           inv_neighbors0, inv_neighbors1, inv_k0, inv_k1, inv_idx0, inv_idx1,
           wn0_W, wn0_b, lin0_W, lin0_b, wn1_W, wn1_b, lin1_W, lin1_b):
    b, n0, _ = xyz0.shape
    m0 = xyz1.shape[1]
    m1 = xyz2.shape[1]

    planes0 = [xyz0[..., d].reshape(-1) for d in range(3)]
    planes0 += [init_feats[..., d].reshape(-1) for d in range(3)]
    idx0 = nei_inds0.transpose(2, 0, 1).reshape(-1)
    g0 = _sc_gather_planar(planes0, idx0, n0, b * m0, m0)
    g0 = g0.reshape(_K, b * m0, 16)
    f1 = _tc_layer(
        g0, None, xyz1.reshape(b * m0, 3), wn0_W, wn0_b,
        _wmajor(lin0_W, 3), lin0_b, cin=3, cout=256, mt=512,
        fused_feat_off=3,
    )

    planes1 = [xyz1[..., d].reshape(-1) for d in range(3)]
    idx1l = nei_inds1.transpose(2, 0, 1).reshape(-1)
    gx1 = _sc_gather_planar(planes1, idx1l, m0, b * m1, m1)
    idx1 = _flat_idx(nei_inds1, m0)
    (gf1,) = _sc_gather([f1], idx1)
    gx1 = gx1.reshape(_K, b * m1, 16)
    gf1 = gf1.reshape(_K, b * m1, 256)
    out = _tc_layer(
        gx1, gf1, xyz2.reshape(b * m1, 3), wn1_W, wn1_b,
        _wmajor(lin1_W, 256), lin1_b, cin=256, cout=1024, mt=128,
    )
    return out.reshape(b, m1, 1024)
